# baseline (speedup 1.0000x reference)
import functools

import jax
import jax.numpy as jnp
from jax import lax
from jax.experimental import pallas as pl
from jax.experimental.pallas import tpu as pltpu
from jax.experimental.pallas import tpu_sc as plsc

NUM_KNNS = 200
NUM_CLASSES = 1000
TEMP = 0.07
TILE = 32
GTILE = 128
NTSEL = 256
NCPAD = 1008
NW = 32
LANES = 16


def _k1_body(f_ref, q_ref, sims_ref, t32_ref):
    f = f_ref[...]
    q = q_ref[...]
    sim = lax.dot_general(f, q, (((1,), (1,)), ((), ())),
                          precision=lax.Precision.HIGHEST)
    sims_ref[...] = sim
    rb, cb = sim.shape
    t32_ref[...] = jnp.max(sim.reshape(rb, cb // TILE, TILE), axis=-1)


def _f32_key(x):
    u = lax.bitcast_convert_type(x, jnp.uint32)
    flip = jnp.where((u >> 31) > 0, jnp.uint32(0xFFFFFFFF),
                     jnp.uint32(0x80000000))
    return u ^ flip


def _key_to_f32(k):
    flip = jnp.where((k >> 31) > 0, jnp.uint32(0x80000000),
                     jnp.uint32(0xFFFFFFFF))
    return lax.bitcast_convert_type(k ^ flip, jnp.float32)


def _k2_body(t32_ref, thr_ref, rowmax_ref):
    t = t32_ref[...]
    rowmax_ref[...] = jnp.max(t, axis=1, keepdims=True)
    keys = _f32_key(t)
    rb = t.shape[0]
    lo = jnp.zeros((rb, 1), jnp.uint32)
    hi = jnp.full((rb, 1), 0xFFFFFFFE, jnp.uint32)

    def body(_, carry):
        lo, hi = carry
        mid = lo + (hi - lo + jnp.uint32(1)) // jnp.uint32(2)
        cnt = jnp.sum((keys >= mid).astype(jnp.int32), axis=1, keepdims=True)
        ge = cnt >= NUM_KNNS
        return (jnp.where(ge, mid, lo), jnp.where(ge, hi, mid - jnp.uint32(1)))

    lo, hi = lax.fori_loop(0, 33, body, (lo, hi))
    thr_ref[...] = _key_to_f32(lo)


def _sc_body(sims2d_hbm, t32_hbm, thr_hbm, rowmax_hbm, qlab_hbm, out_hbm,
             labels_v, t32row_v, tilesel_v, rowsel_v, gathered_v, thr_v,
             rm_v, scores_v, preds_v, sem):
    nt = t32row_v.shape[0]
    rpw = preds_v.shape[0]
    wid = lax.axis_index("s") * 2 + lax.axis_index("c")
    base = wid * rpw
    lane = lax.iota(jnp.int32, LANES)

    pltpu.sync_copy(qlab_hbm, labels_v)
    pltpu.sync_copy(thr_hbm.at[pl.ds(base, rpw)], thr_v.at[pl.ds(0, rpw)])
    pltpu.sync_copy(rowmax_hbm.at[pl.ds(base, rpw)], rm_v.at[pl.ds(0, rpw)])

    def row_body(i, _):
        r = base + i
        pltpu.sync_copy(t32_hbm.at[r], t32row_v)
        thr_sc = thr_v[pl.ds(i, LANES)][0]
        rm_sc = rm_v[pl.ds(i, LANES)][0]
        thr_s = jnp.full((LANES,), thr_sc)
        rowbase = r * nt

        def zsel(j, _):
            tilesel_v[pl.ds(j * LANES, LANES)] = jnp.zeros((LANES,), jnp.int32)
            return 0

        lax.fori_loop(0, NTSEL // LANES, zsel, 0)

        def zsc(j, _):
            scores_v[pl.ds(j * LANES, LANES)] = jnp.zeros((LANES,), jnp.float32)
            return 0

        lax.fori_loop(0, NCPAD // LANES, zsc, 0)

        def sel(j, cnt):
            tv = t32row_v[pl.ds(j * LANES, LANES)]
            m = (tv >= thr_s) & (cnt < NTSEL - LANES)
            gidx = rowbase + j * LANES + lane
            start = jnp.minimum(cnt, NTSEL - LANES)
            plsc.store_compressed(tilesel_v.at[pl.ds(start, LANES)], gidx,
                                  mask=m)
            return cnt + jnp.sum(m.astype(jnp.int32))

        cnt = lax.fori_loop(0, nt // LANES, sel, jnp.int32(0))

        def conv(j, _):
            sl = pl.ds(j * LANES, LANES)
            rowsel_v[sl] = lax.shift_right_logical(tilesel_v[sl], 2)
            return 0

        lax.fori_loop(0, NTSEL // LANES, conv, 0)

        pltpu.async_copy(sims2d_hbm.at[rowsel_v], gathered_v, sem).wait()

        def vote(t, _):
            gid = tilesel_v[pl.ds(t, LANES)][0]
            sub = (gid & 3) * TILE
            colbase = (gid - rowbase) * TILE
            for h in range(TILE // LANES):
                vals = gathered_v[t, pl.ds(sub + h * LANES, LANES)]
                colv = colbase + (h * LANES) + lane
                m = vals >= thr_s
                w = jnp.exp((vals - rm_sc) * (1.0 / TEMP))
                labs = plsc.load_gather(labels_v, [colv])
                plsc.addupdate_scatter(scores_v, [labs], w, mask=m)
            return 0

        lax.fori_loop(0, cnt, vote, 0)

        def am(c, carry):
            bv, bi = carry
            v = scores_v[pl.ds(c * LANES, LANES)]
            idxv = c * LANES + lane
            m = v > bv
            return (jnp.where(m, v, bv), jnp.where(m, idxv, bi))

        bv, bi = lax.fori_loop(0, NCPAD // LANES, am,
                               (jnp.full((LANES,), -1.0, jnp.float32),
                                jnp.zeros((LANES,), jnp.int32)))
        mx = jnp.max(bv)
        pred = jnp.min(jnp.where(bv == mx, bi, jnp.int32(1 << 20)))
        plsc.store_scatter(preds_v, [jnp.full((LANES,), i, jnp.int32)],
                           jnp.full((LANES,), pred, jnp.int32),
                           mask=(lane == 0))
        return 0

    lax.fori_loop(0, rpw, row_body, 0)
    pltpu.sync_copy(preds_v, out_hbm.at[pl.ds(base, rpw)])


def _k4_body(pred_ref, lab_ref, out_ref):
    matches = (pred_ref[0, :] == lab_ref[0, :]).astype(jnp.float32)
    out_ref[...] = jnp.sum(matches).reshape(1, 1)


def kernel(features, labels, queue_features, queue_labels, train):
    b, d = features.shape
    k = queue_features.shape[0]
    rb = min(256, b)
    cb = 4096 if k % 4096 == 0 else k
    nt = k // TILE
    rpw = b // NW

    sims, t32 = pl.pallas_call(
        _k1_body,
        grid=(b // rb, k // cb),
        in_specs=[
            pl.BlockSpec((rb, d), lambda i, j: (i, 0)),
            pl.BlockSpec((cb, d), lambda i, j: (j, 0)),
        ],
        out_specs=[
            pl.BlockSpec((rb, cb), lambda i, j: (i, j)),
            pl.BlockSpec((rb, cb // TILE), lambda i, j: (i, j)),
        ],
        out_shape=[
            jax.ShapeDtypeStruct((b, k), jnp.float32),
            jax.ShapeDtypeStruct((b, nt), jnp.float32),
        ],
    )(features, queue_features)

    ng = k // GTILE
    thr, rowmax = pl.pallas_call(
        _k2_body,
        grid=(b // rb,),
        in_specs=[pl.BlockSpec((rb, nt), lambda i: (i, 0))],
        out_specs=[
            pl.BlockSpec((rb, 1), lambda i: (i, 0)),
            pl.BlockSpec((rb, 1), lambda i: (i, 0)),
        ],
        out_shape=[
            jax.ShapeDtypeStruct((b, 1), jnp.float32),
            jax.ShapeDtypeStruct((b, 1), jnp.float32),
        ],
    )(t32)

    mesh = plsc.VectorSubcoreMesh(core_axis_name="c", subcore_axis_name="s",
                                  num_cores=2, num_subcores=16)
    preds = pl.kernel(
        _sc_body,
        out_type=jax.ShapeDtypeStruct((b,), jnp.int32),
        mesh=mesh,
        compiler_params=pltpu.CompilerParams(needs_layout_passes=False),
        scratch_types=[
            pltpu.VMEM((k,), jnp.int32),
            pltpu.VMEM((nt,), jnp.float32),
            pltpu.VMEM((NTSEL + LANES,), jnp.int32),
            pltpu.VMEM((NTSEL,), jnp.int32),
            pltpu.VMEM((NTSEL, GTILE), jnp.float32),
            pltpu.VMEM((rpw + LANES,), jnp.float32),
            pltpu.VMEM((rpw + LANES,), jnp.float32),
            pltpu.VMEM((NCPAD,), jnp.float32),
            pltpu.VMEM((rpw,), jnp.int32),
            pltpu.SemaphoreType.DMA,
        ],
    )(sims.reshape(b * ng, GTILE), t32, thr.reshape(b), rowmax.reshape(b),
      queue_labels)

    nsum = pl.pallas_call(
        _k4_body,
        in_specs=[
            pl.BlockSpec((1, b), lambda: (0, 0)),
            pl.BlockSpec((1, b), lambda: (0, 0)),
        ],
        out_specs=pl.BlockSpec((1, 1), lambda: (0, 0)),
        out_shape=jax.ShapeDtypeStruct((1, 1), jnp.float32),
    )(preds.reshape(1, b), labels.reshape(1, b))

    acc = nsum[0, 0] / b
    return acc * jnp.asarray(train, dtype=acc.dtype)


# speedup vs baseline: 2.2741x; 2.2741x over previous
import functools

import jax
import jax.numpy as jnp
from jax import lax
from jax.experimental import pallas as pl
from jax.experimental.pallas import tpu as pltpu
from jax.experimental.pallas import tpu_sc as plsc

NUM_KNNS = 200
NUM_CLASSES = 1000
TEMP = 0.07
TILE = 32
GTILE = 128
NTSEL = 256
NCPAD = 1024
NW = 32
LANES = 16


def _k1_body(f_ref, q_ref, sims_ref, t32_ref):
    f = f_ref[...]
    q = q_ref[...]
    sim = lax.dot_general(f, q, (((1,), (1,)), ((), ())),
                          precision=lax.Precision.HIGHEST)
    sims_ref[...] = sim
    rb, cb = sim.shape
    t32_ref[...] = jnp.max(sim.reshape(rb, cb // TILE, TILE), axis=-1)


def _f32_key(x):
    u = lax.bitcast_convert_type(x, jnp.uint32)
    flip = jnp.where((u >> 31) > 0, jnp.uint32(0xFFFFFFFF),
                     jnp.uint32(0x80000000))
    return u ^ flip


def _key_to_f32(k):
    flip = jnp.where((k >> 31) > 0, jnp.uint32(0x80000000),
                     jnp.uint32(0xFFFFFFFF))
    return lax.bitcast_convert_type(k ^ flip, jnp.float32)


def _k2_body(t32_ref, thr_ref, rowmax_ref):
    t = t32_ref[...]
    rowmax_ref[...] = jnp.max(t, axis=1, keepdims=True)
    keys = _f32_key(t)
    rb = t.shape[0]
    lo = jnp.zeros((rb, 1), jnp.uint32)
    hi = jnp.full((rb, 1), 0xFFFFFFFE, jnp.uint32)

    def body(_, carry):
        lo, hi = carry
        mid = lo + (hi - lo + jnp.uint32(1)) // jnp.uint32(2)
        cnt = jnp.sum((keys >= mid).astype(jnp.int32), axis=1, keepdims=True)
        ge = cnt >= NUM_KNNS
        return (jnp.where(ge, mid, lo), jnp.where(ge, hi, mid - jnp.uint32(1)))

    lo, hi = lax.fori_loop(0, 33, body, (lo, hi))
    thr_ref[...] = _key_to_f32(lo)


SEL_UNROLL = 8


def _sc_body(sims2d_hbm, t32_hbm, thr_hbm, rowmax_hbm, qlab_hbm, out_hbm,
             labels_v, t32row2_v, tilesel_v, rowsel_v, gathered_v, thr_v,
             rm_v, scores_v, preds_v, sem_t, sem_g):
    nt = t32row2_v.shape[1]
    rpw = preds_v.shape[0]
    wid = lax.axis_index("s") * 2 + lax.axis_index("c")
    base = wid * rpw
    lane = lax.iota(jnp.int32, LANES)

    pltpu.sync_copy(qlab_hbm, labels_v)
    pltpu.sync_copy(thr_hbm.at[pl.ds(base, rpw)], thr_v.at[pl.ds(0, rpw)])
    pltpu.sync_copy(rowmax_hbm.at[pl.ds(base, rpw)], rm_v.at[pl.ds(0, rpw)])
    pltpu.async_copy(t32_hbm.at[base], t32row2_v.at[0], sem_t)

    def row_body(i, _):
        r = base + i
        slot = lax.rem(i, 2)
        thr_sc = thr_v[pl.ds(i, LANES)][0]
        rm_sc = rm_v[pl.ds(i, LANES)][0]
        thr_s = jnp.full((LANES,), thr_sc)
        rowbase = r * nt

        pltpu.make_async_copy(t32_hbm.at[r], t32row2_v.at[slot], sem_t).wait()

        def sel(j0, cnt):
            for u in range(SEL_UNROLL):
                j = j0 * SEL_UNROLL + u
                tv = t32row2_v[slot, pl.ds(j * LANES, LANES)]
                m = (tv >= thr_s) & (cnt < NTSEL - LANES)
                gidx = rowbase + j * LANES + lane
                start = jnp.minimum(cnt, NTSEL - LANES)
                plsc.store_compressed(tilesel_v.at[pl.ds(start, LANES)],
                                      gidx, mask=m)
                plsc.store_compressed(rowsel_v.at[pl.ds(start, LANES)],
                                      gidx, mask=m)
                cnt = cnt + jnp.sum(m.astype(jnp.int32))
            return cnt

        cnt = lax.fori_loop(0, nt // (LANES * SEL_UNROLL), sel, jnp.int32(0))

        def zpad(j0, _):
            start = jnp.minimum(cnt + j0 * LANES, NTSEL - LANES)
            rowsel_v[pl.ds(start, LANES)] = jnp.zeros((LANES,), jnp.int32)
            return 0

        lax.fori_loop(0, (NTSEL - cnt + LANES - 1) // LANES, zpad, 0)

        gather = pltpu.async_copy(sims2d_hbm.at[rowsel_v], gathered_v, sem_g)
        rn = jnp.minimum(r + 1, base + rpw - 1)
        pltpu.async_copy(t32_hbm.at[rn], t32row2_v.at[1 - slot], sem_t)
        for j in range(NCPAD // LANES):
            scores_v[pl.ds(j * LANES, LANES)] = jnp.zeros((LANES,),
                                                          jnp.float32)
        gather.wait()

        def vote(t, _):
            gid = tilesel_v[pl.ds(t, LANES)][0]
            colbase = (gid - rowbase) * TILE
            for h in range(TILE // LANES):
                vals = gathered_v[t, pl.ds(h * LANES, LANES)]
                colv = colbase + (h * LANES) + lane
                m = vals >= thr_s
                w = jnp.exp((vals - rm_sc) * (1.0 / TEMP))
                labs = plsc.load_gather(labels_v, [colv])
                plsc.addupdate_scatter(scores_v, [labs], w, mask=m)
            return 0

        lax.fori_loop(0, cnt, vote, 0)

        def am(c, carry):
            bv, bi = carry
            v = scores_v[pl.ds(c * LANES, LANES)]
            idxv = c * LANES + lane
            m = v > bv
            return (jnp.where(m, v, bv), jnp.where(m, idxv, bi))

        bv, bi = lax.fori_loop(0, NCPAD // LANES, am,
                               (jnp.full((LANES,), -1.0, jnp.float32),
                                jnp.zeros((LANES,), jnp.int32)))
        mx = jnp.max(bv)
        pred = jnp.min(jnp.where(bv == mx, bi, jnp.int32(1 << 20)))
        plsc.store_scatter(preds_v, [jnp.full((LANES,), i, jnp.int32)],
                           jnp.full((LANES,), pred, jnp.int32),
                           mask=(lane == 0))
        return 0

    lax.fori_loop(0, rpw, row_body, 0)
    pltpu.make_async_copy(t32_hbm.at[base + rpw - 1],
                          t32row2_v.at[lax.rem(rpw, 2)], sem_t).wait()
    pltpu.sync_copy(preds_v, out_hbm.at[pl.ds(base, rpw)])


def _k4_body(pred_ref, lab_ref, out_ref):
    matches = (pred_ref[0, :] == lab_ref[0, :]).astype(jnp.float32)
    out_ref[...] = jnp.sum(matches).reshape(1, 1)


def kernel(features, labels, queue_features, queue_labels, train):
    b, d = features.shape
    k = queue_features.shape[0]
    rb = min(256, b)
    cb = 4096 if k % 4096 == 0 else k
    nt = k // TILE
    rpw = b // NW

    sims, t32 = pl.pallas_call(
        _k1_body,
        grid=(b // rb, k // cb),
        in_specs=[
            pl.BlockSpec((rb, d), lambda i, j: (i, 0)),
            pl.BlockSpec((cb, d), lambda i, j: (j, 0)),
        ],
        out_specs=[
            pl.BlockSpec((rb, cb), lambda i, j: (i, j)),
            pl.BlockSpec((rb, cb // TILE), lambda i, j: (i, j)),
        ],
        out_shape=[
            jax.ShapeDtypeStruct((b, k), jnp.float32),
            jax.ShapeDtypeStruct((b, nt), jnp.float32),
        ],
    )(features, queue_features)

    ng = k // GTILE
    thr, rowmax = pl.pallas_call(
        _k2_body,
        grid=(b // rb,),
        in_specs=[pl.BlockSpec((rb, nt), lambda i: (i, 0))],
        out_specs=[
            pl.BlockSpec((rb, 1), lambda i: (i, 0)),
            pl.BlockSpec((rb, 1), lambda i: (i, 0)),
        ],
        out_shape=[
            jax.ShapeDtypeStruct((b, 1), jnp.float32),
            jax.ShapeDtypeStruct((b, 1), jnp.float32),
        ],
    )(t32)

    mesh = plsc.VectorSubcoreMesh(core_axis_name="c", subcore_axis_name="s",
                                  num_cores=2, num_subcores=16)
    preds = pl.kernel(
        _sc_body,
        out_type=jax.ShapeDtypeStruct((b,), jnp.int32),
        mesh=mesh,
        compiler_params=pltpu.CompilerParams(needs_layout_passes=False,
                                             use_tc_tiling_on_sc=False),
        scratch_types=[
            pltpu.VMEM((k,), jnp.int32),
            pltpu.VMEM((2, nt), jnp.float32),
            pltpu.VMEM((NTSEL + LANES,), jnp.int32),
            pltpu.VMEM((NTSEL,), jnp.int32),
            pltpu.VMEM((NTSEL, TILE), jnp.float32),
            pltpu.VMEM((rpw + LANES,), jnp.float32),
            pltpu.VMEM((rpw + LANES,), jnp.float32),
            pltpu.VMEM((NCPAD,), jnp.float32),
            pltpu.VMEM((rpw,), jnp.int32),
            pltpu.SemaphoreType.DMA,
            pltpu.SemaphoreType.DMA,
        ],
    )(sims.reshape(b * nt, TILE), t32, thr.reshape(b), rowmax.reshape(b),
      queue_labels)

    nsum = pl.pallas_call(
        _k4_body,
        in_specs=[
            pl.BlockSpec((1, b), lambda: (0, 0)),
            pl.BlockSpec((1, b), lambda: (0, 0)),
        ],
        out_specs=pl.BlockSpec((1, 1), lambda: (0, 0)),
        out_shape=jax.ShapeDtypeStruct((1, 1), jnp.float32),
    )(preds.reshape(1, b), labels.reshape(1, b))

    acc = nsum[0, 0] / b
    return acc * jnp.asarray(train, dtype=acc.dtype)
